# Initial kernel scaffold; baseline (speedup 1.0000x reference)
import jax
import jax.numpy as jnp
from jax.experimental import pallas as pl


def kernel(dec_x, dec_pc, enc_x, enc_pc, params):
    raise NotImplementedError("write your pallas kernel here")


# speedup vs baseline: 10.9749x; 10.9749x over previous
import functools

import jax
import jax.numpy as jnp
from jax import lax
from jax.experimental import pallas as pl
from jax.experimental.pallas import tpu as pltpu
from jax.experimental.pallas import tpu_sc as plsc

B, N1, N2 = 2, 1024, 1024
IN1, IN2, DIM = 256, 512, 256
POS_HID, ATTN_HID, KNN = 64, 64, 16
NPC = N1 + N2
TW = DIM + ATTN_HID + POS_HID
NW = 32
F32 = jnp.float32


def _dot(a, b):
    return jax.lax.dot_general(a, b, (((a.ndim - 1,), (0,)), ((), ())),
                               preferred_element_type=F32)


def _table_body(dec_x_ref, enc_x_ref, pc_ref, p1w, p1b, p2w, p2b, vw, vb,
                kw2, kb2, t_ref):
    j = pl.program_id(1)

    def emit(x):
        tv = _dot(x, vw[...]) + vb[...]
        tk = _dot(x, kw2[...]) + kb2[...]
        pad = jnp.zeros((tv.shape[0], 48), F32)
        t_ref[0] = jnp.concatenate([tv, tk, pc_ref[0], pad], axis=-1)

    @pl.when(j < 4)
    def _():
        emit(_dot(dec_x_ref[0], p1w[...]) + p1b[...])

    @pl.when(j >= 4)
    def _():
        emit(_dot(enc_x_ref[0], p2w[...]) + p2b[...])


def _build_table(dec_x, enc_x, pc16, p):
    blk = 256
    return pl.pallas_call(
        _table_body,
        grid=(B, NPC // blk),
        in_specs=[
            pl.BlockSpec((1, blk, IN1), lambda b, j: (b, jnp.minimum(j, 3), 0)),
            pl.BlockSpec((1, blk, IN2), lambda b, j: (b, jnp.maximum(j - 4, 0), 0)),
            pl.BlockSpec((1, blk, 16), lambda b, j: (b, j, 0)),
            pl.BlockSpec((IN1, DIM), lambda b, j: (0, 0)),
            pl.BlockSpec((1, DIM), lambda b, j: (0, 0)),
            pl.BlockSpec((IN2, DIM), lambda b, j: (0, 0)),
            pl.BlockSpec((1, DIM), lambda b, j: (0, 0)),
            pl.BlockSpec((DIM, DIM), lambda b, j: (0, 0)),
            pl.BlockSpec((1, DIM), lambda b, j: (0, 0)),
            pl.BlockSpec((DIM, ATTN_HID), lambda b, j: (0, 0)),
            pl.BlockSpec((1, ATTN_HID), lambda b, j: (0, 0)),
        ],
        out_specs=pl.BlockSpec((1, blk, TW), lambda b, j: (b, j, 0)),
        out_shape=jax.ShapeDtypeStruct((B, NPC, TW), F32),
    )(dec_x, enc_x, pc16, *p)


def _fps_body(pcp_ref, out_ref):
    pcp = pcp_ref[...]
    gidx = (lax.broadcasted_iota(jnp.int32, (B, 8, 256), 1) * 256
            + lax.broadcasted_iota(jnp.int32, (B, 8, 256), 2))
    BIGI = jnp.int32(1 << 30)

    def red2(x, fn):
        return fn(fn(x, axis=2, keepdims=True), axis=1, keepdims=True)

    def step(t, carry):
        dist, far = carry
        out_ref[pl.ds(t, 1), :] = jnp.reshape(far, (1, B))
        eq = gidx == far
        d = None
        for c in range(3):
            cc = red2(jnp.where(eq, pcp[:, c], 0.0), jnp.sum)
            dc = (pcp[:, c] - cc) ** 2
            d = dc if d is None else d + dc
        dist = jnp.minimum(dist, d)
        mx = red2(dist, jnp.max)
        far = red2(jnp.where(dist == mx, gidx, BIGI), jnp.min)
        return dist, far

    init = (jnp.full((B, 8, 256), 1e10, F32), jnp.zeros((B, 1, 1), jnp.int32))
    lax.fori_loop(0, N1, step, init)


def _fps(pc_fps):
    return pl.pallas_call(
        _fps_body,
        in_specs=[pl.BlockSpec((B, 3, 8, 256), lambda: (0, 0, 0, 0))],
        out_specs=pl.BlockSpec((N1, B), lambda: (0, 0)),
        out_shape=jax.ShapeDtypeStruct((N1, B), jnp.int32),
    )(pc_fps)


def _sc_mesh():
    return dict(mesh=plsc.VectorSubcoreMesh(core_axis_name="c",
                                            subcore_axis_name="s"))


def _wid():
    return lax.axis_index("s") * 2 + lax.axis_index("c")


def _gather_fps(t_flat, pc128_flat, fidx_flat):
    rpw = (B * N1) // NW

    @functools.partial(
        pl.kernel, **_sc_mesh(),
        out_type=[jax.ShapeDtypeStruct((B * N1, TW), F32),
                  jax.ShapeDtypeStruct((B * N1, 128), F32)],
        scratch_types=[pltpu.VMEM((rpw,), jnp.int32),
                       pltpu.VMEM((rpw, TW), F32),
                       pltpu.VMEM((rpw, 128), F32),
                       pltpu.SemaphoreType.DMA,
                       pltpu.SemaphoreType.DMA],
    )
    def k(t_hbm, pc_hbm, fidx_hbm, tf_hbm, pcf_hbm, idx_v, buft, bufp,
          semt, semp):
        w = _wid()
        b = w // 16
        base = w * rpw
        pltpu.sync_copy(fidx_hbm.at[pl.ds(base, rpw)], idx_v)
        for i in range(rpw // 16):
            idx_v[pl.ds(i * 16, 16)] = idx_v[pl.ds(i * 16, 16)] + b * NPC
        cpt = pltpu.async_copy(t_hbm.at[idx_v], buft, semt)
        cpp = pltpu.async_copy(pc_hbm.at[idx_v], bufp, semp)
        cpt.wait()
        pltpu.sync_copy(buft, tf_hbm.at[pl.ds(base, rpw)])
        cpp.wait()
        pltpu.sync_copy(bufp, pcf_hbm.at[pl.ds(base, rpw)])

    return k(t_flat, pc128_flat, fidx_flat)


def _gather_table(tf_flat, knn_flat):
    rpw = (B * N1 * KNN) // NW
    chunk = 128
    nch = rpw // chunk

    @functools.partial(
        pl.kernel, **_sc_mesh(),
        out_type=jax.ShapeDtypeStruct((B * N1 * KNN, TW), F32),
        scratch_types=[pltpu.VMEM((rpw,), jnp.int32),
                       pltpu.VMEM((chunk, TW), F32),
                       pltpu.VMEM((chunk, TW), F32),
                       pltpu.SemaphoreType.DMA,
                       pltpu.SemaphoreType.DMA],
    )
    def k(tf_hbm, knn_hbm, out_hbm, aidx_v, buf0, buf1, sem0, sem1):
        w = _wid()
        b = w // 16
        base = w * rpw
        pltpu.sync_copy(knn_hbm.at[pl.ds(base, rpw)], aidx_v)

        def shift(i, carry):
            aidx_v[pl.ds(i * 16, 16)] = aidx_v[pl.ds(i * 16, 16)] + b * N1
            return carry

        lax.fori_loop(0, rpw // 16, shift, 0)

        bufs, sems = (buf0, buf1), (sem0, sem1)
        prev = pltpu.async_copy(tf_hbm.at[aidx_v.at[pl.ds(0, chunk)]], buf0, sem0)
        for c in range(1, nch):
            cur = pltpu.async_copy(tf_hbm.at[aidx_v.at[pl.ds(c * chunk, chunk)]],
                                   bufs[c % 2], sems[c % 2])
            prev.wait()
            pltpu.sync_copy(bufs[(c - 1) % 2],
                            out_hbm.at[pl.ds(base + (c - 1) * chunk, chunk)])
            prev = cur
        prev.wait()
        pltpu.sync_copy(bufs[(nch - 1) % 2],
                        out_hbm.at[pl.ds(base + (nch - 1) * chunk, chunk)])

    return k(tf_flat, knn_flat)


def _knn_body(pc1_ref, pcfT_ref, knn_ref):
    pc1 = pc1_ref[0]
    pcfT = pcfT_ref[0]
    aa = ((pc1[:, 0:1] * pc1[:, 0:1] + pc1[:, 1:2] * pc1[:, 1:2])
          + pc1[:, 2:3] * pc1[:, 2:3])
    bb = ((pcfT[0:1, :] * pcfT[0:1, :] + pcfT[1:2, :] * pcfT[1:2, :])
          + pcfT[2:3, :] * pcfT[2:3, :])
    mat = aa - 2.0 * _dot(pc1, pcfT) + bb
    lane = lax.broadcasted_iota(jnp.int32, (N1, N1), 1)
    BIGI = jnp.int32(1 << 30)
    for t in range(KNN):
        mn = jnp.min(mat, axis=-1, keepdims=True)
        idx = jnp.min(jnp.where(mat == mn, lane, BIGI), axis=-1, keepdims=True)
        knn_ref[0, :, t:t + 1] = idx
        mat = jnp.where(lane == idx, jnp.inf, mat)


def _knn(dec_pc16, pcfT):
    return pl.pallas_call(
        _knn_body,
        grid=(B,),
        in_specs=[pl.BlockSpec((1, N1, 16), lambda b: (b, 0, 0)),
                  pl.BlockSpec((1, 16, N1), lambda b: (b, 0, 0))],
        out_specs=pl.BlockSpec((1, N1, KNN), lambda b: (b, 0, 0)),
        out_shape=jax.ShapeDtypeStruct((B, N1, KNN), jnp.int32),
    )(dec_pc16, pcfT)


def _stats1_body(gp_ref, pc_ref, w1p, b1, ssum_ref, ssq_ref):
    first = (pl.program_id(0) == 0) & (pl.program_id(1) == 0)
    pc1 = jnp.reshape(jnp.broadcast_to(pc_ref[0][:, None, :], (128, KNN, 16)),
                      (128 * KNN, 16))
    h = _dot(gp_ref[0][:, POS_HID:POS_HID + 16] - pc1, w1p[...]) + b1[...]
    ones = jnp.ones((1, h.shape[0]), F32)
    s = _dot(ones, h)
    q = _dot(ones, h * h)

    @pl.when(first)
    def _():
        ssum_ref[...] = s
        ssq_ref[...] = q

    @pl.when(jnp.logical_not(first))
    def _():
        ssum_ref[...] += s
        ssq_ref[...] += q


def _stats1(g, dec_pc16, w1p, b1r):
    return pl.pallas_call(
        _stats1_body,
        grid=(B, 8),
        in_specs=[
            pl.BlockSpec((1, 2048, 128), lambda b, i: (b, i, 2)),
            pl.BlockSpec((1, 128, 16), lambda b, i: (b, i, 0)),
            pl.BlockSpec((16, POS_HID), lambda b, i: (0, 0)),
            pl.BlockSpec((1, POS_HID), lambda b, i: (0, 0)),
        ],
        out_specs=[pl.BlockSpec((1, POS_HID), lambda b, i: (0, 0)),
                   pl.BlockSpec((1, POS_HID), lambda b, i: (0, 0))],
        out_shape=[jax.ShapeDtypeStruct((1, POS_HID), F32),
                   jax.ShapeDtypeStruct((1, POS_HID), F32)],
    )(g, dec_pc16, w1p, b1r)


def _mid_body(g_ref, dx_ref, pc_ref, wq3, bq3, w1p, b1, a1, c1, w2, b2, wpa,
              cpa, vp_ref, h2_ref, ssum_ref, ssq_ref):
    first = (pl.program_id(0) == 0) & (pl.program_id(1) == 0)
    g = g_ref[0]

    def rep16(x):
        h = x.shape[-1]
        return jnp.reshape(jnp.broadcast_to(x[:, None, :], (128, KNN, h)),
                           (128 * KNN, h))

    knn_pc = g[:, DIM + ATTN_HID:DIM + ATTN_HID + 16] - rep16(pc_ref[0])
    h1pre = _dot(knn_pc, w1p[...]) + b1[...]
    h1 = jnp.maximum(h1pre * a1[...] + c1[...], 0.0)
    pos = _dot(h1, w2[...]) + b2[...]
    vp_ref[0] = g[:, :DIM] + pos
    q3 = rep16(_dot(dx_ref[0], wq3[...]) + bq3[...])
    h2 = g[:, DIM:DIM + ATTN_HID] - q3 + _dot(h1, wpa[...]) + cpa[...]
    h2_ref[0] = h2
    ones = jnp.ones((1, h2.shape[0]), F32)
    s = _dot(ones, h2)
    q = _dot(ones, h2 * h2)

    @pl.when(first)
    def _():
        ssum_ref[...] = s
        ssq_ref[...] = q

    @pl.when(jnp.logical_not(first))
    def _():
        ssum_ref[...] += s
        ssq_ref[...] += q


def _mid(g, dec_x, dec_pc16, consts):
    return pl.pallas_call(
        _mid_body,
        grid=(B, 8),
        in_specs=[
            pl.BlockSpec((1, 2048, TW), lambda b, i: (b, i, 0)),
            pl.BlockSpec((1, 128, IN1), lambda b, i: (b, i, 0)),
            pl.BlockSpec((1, 128, 16), lambda b, i: (b, i, 0)),
            pl.BlockSpec((IN1, ATTN_HID), lambda b, i: (0, 0)),
            pl.BlockSpec((1, ATTN_HID), lambda b, i: (0, 0)),
            pl.BlockSpec((16, POS_HID), lambda b, i: (0, 0)),
            pl.BlockSpec((1, POS_HID), lambda b, i: (0, 0)),
            pl.BlockSpec((1, POS_HID), lambda b, i: (0, 0)),
            pl.BlockSpec((1, POS_HID), lambda b, i: (0, 0)),
            pl.BlockSpec((POS_HID, DIM), lambda b, i: (0, 0)),
            pl.BlockSpec((1, DIM), lambda b, i: (0, 0)),
            pl.BlockSpec((POS_HID, ATTN_HID), lambda b, i: (0, 0)),
            pl.BlockSpec((1, ATTN_HID), lambda b, i: (0, 0)),
        ],
        out_specs=[pl.BlockSpec((1, 2048, DIM), lambda b, i: (b, i, 0)),
                   pl.BlockSpec((1, 2048, ATTN_HID), lambda b, i: (b, i, 0)),
                   pl.BlockSpec((1, ATTN_HID), lambda b, i: (0, 0)),
                   pl.BlockSpec((1, ATTN_HID), lambda b, i: (0, 0))],
        out_shape=[jax.ShapeDtypeStruct((B, N1 * KNN, DIM), F32),
                   jax.ShapeDtypeStruct((B, N1 * KNN, ATTN_HID), F32),
                   jax.ShapeDtypeStruct((1, ATTN_HID), F32),
                   jax.ShapeDtypeStruct((1, ATTN_HID), F32)],
    )(g, dec_x, dec_pc16, *consts)


def _fin_body(vp_ref, h2_ref, dx_ref, a2, c2k, aw2, ab2, p1w, p1b, p2w, p2b,
              dec_ref, enc_ref):
    h2 = jnp.maximum(h2_ref[0] * a2[...] + c2k[...], 0.0)
    attn = _dot(h2, aw2[...]) + ab2[...]
    a3 = jnp.reshape(attn, (128, KNN, DIM))
    m = jnp.max(a3, axis=1, keepdims=True)
    e = jnp.exp(a3 - m)
    p = e / jnp.sum(e, axis=1, keepdims=True)
    vp3 = jnp.reshape(vp_ref[0], (128, KNN, DIM))
    out = jnp.sum(vp3 * p, axis=1)
    dec_ref[0] = _dot(out, p1w[...]) + p1b[...] + dx_ref[0]
    enc_ref[0] = _dot(out, p2w[...]) + p2b[...]


def _fin(vp, h2, dec_x, consts):
    return pl.pallas_call(
        _fin_body,
        grid=(B, 8),
        in_specs=[
            pl.BlockSpec((1, 2048, DIM), lambda b, i: (b, i, 0)),
            pl.BlockSpec((1, 2048, ATTN_HID), lambda b, i: (b, i, 0)),
            pl.BlockSpec((1, 128, IN1), lambda b, i: (b, i, 0)),
            pl.BlockSpec((1, ATTN_HID), lambda b, i: (0, 0)),
            pl.BlockSpec((1, ATTN_HID), lambda b, i: (0, 0)),
            pl.BlockSpec((ATTN_HID, DIM), lambda b, i: (0, 0)),
            pl.BlockSpec((1, DIM), lambda b, i: (0, 0)),
            pl.BlockSpec((DIM, IN1), lambda b, i: (0, 0)),
            pl.BlockSpec((1, IN1), lambda b, i: (0, 0)),
            pl.BlockSpec((DIM, IN2), lambda b, i: (0, 0)),
            pl.BlockSpec((1, IN2), lambda b, i: (0, 0)),
        ],
        out_specs=[pl.BlockSpec((1, 128, IN1), lambda b, i: (b, i, 0)),
                   pl.BlockSpec((1, 128, IN2), lambda b, i: (b, i, 0))],
        out_shape=[jax.ShapeDtypeStruct((B, N1, IN1), F32),
                   jax.ShapeDtypeStruct((B, N1, IN2), F32)],
    )(vp, h2, dec_x, *consts)


def kernel(dec_x, dec_pc, enc_x, enc_pc, params):
    p = params
    A1 = p['attn_W1']
    kw2 = p['k_W'] @ A1
    kb2 = p['k_b'] @ A1
    qa = p['q_W'] @ A1
    wq3 = p['pre1_W'] @ qa
    bq3 = p['pre1_b'] @ qa + p['q_b'] @ A1
    wpa = p['pos_W2'] @ A1
    cpa = p['pos_b2'] @ A1 + p['attn_b1']
    w1p = jnp.zeros((16, POS_HID), F32).at[:3].set(p['pos_W1'])

    pc = jnp.concatenate([dec_pc, enc_pc], axis=1)
    pc16 = jnp.pad(pc, ((0, 0), (0, 0), (0, 13)))
    dec_pc16 = jnp.pad(dec_pc, ((0, 0), (0, 0), (0, 13)))
    pc_fps = jnp.transpose(pc, (0, 2, 1)).reshape(B, 3, 8, 256)

    def r2(v):
        return v[None, :]

    t = _build_table(dec_x, enc_x, pc16,
                     (p['pre1_W'], r2(p['pre1_b']), p['pre2_W'], r2(p['pre2_b']),
                      p['v_W'], r2(p['v_b']), kw2, r2(kb2)))

    fidx_t = _fps(pc_fps)
    fidx_flat = jnp.transpose(fidx_t).reshape(-1)

    pc128 = jnp.pad(pc, ((0, 0), (0, 0), (0, 125)))
    tf, pcf = _gather_fps(t.reshape(B * NPC, TW), pc128.reshape(B * NPC, 128),
                          fidx_flat)
    pcfT = jnp.transpose(pcf[:, :16].reshape(B, N1, 16), (0, 2, 1))

    knn = _knn(dec_pc16, pcfT)

    g = _gather_table(tf, knn.reshape(-1)).reshape(B, N1 * KNN, TW)

    s1, q1 = _stats1(g, dec_pc16, w1p, r2(p['pos_b1']))
    n = float(B * N1 * KNN)
    mu1 = s1 / n
    var1 = q1 / n - mu1 * mu1
    a1 = p['pos_g'][None, :] * lax.rsqrt(var1 + 1e-5)
    c1 = p['pos_be'][None, :] - mu1 * a1

    vp, h2, s2, q2 = _mid(g, dec_x, dec_pc16,
                          (wq3, r2(bq3), w1p, r2(p['pos_b1']), a1, c1,
                           p['pos_W2'], r2(p['pos_b2']), wpa, r2(cpa)))
    mu2 = s2 / n
    var2 = q2 / n - mu2 * mu2
    a2 = p['attn_g'][None, :] * lax.rsqrt(var2 + 1e-5)
    c2k = p['attn_be'][None, :] - mu2 * a2

    dec_out, enc_out = _fin(vp, h2, dec_x,
                            (a2, c2k, p['attn_W2'], r2(p['attn_b2']),
                             p['post1_W'], r2(p['post1_b']),
                             p['post2_W'], r2(p['post2_b'])))
    return (dec_out, dec_pc, enc_out, enc_pc)
